# Initial kernel scaffold; baseline (speedup 1.0000x reference)
import functools

import jax
import jax.numpy as jnp
from jax import lax
from jax.experimental import pallas as pl
from jax.experimental.pallas import tpu as pltpu
from jax.experimental.pallas import tpu_sc as plsc

NUM_CORES = 2
NUM_SUBCORES = 16
NW = NUM_CORES * NUM_SUBCORES
CHUNK = 128


def kernel(indices, table):
    B, L = indices.shape
    V, D = table.shape
    N = B * L
    assert N % (NW * CHUNK) == 0
    per_w = N // NW
    n_chunks = per_w // CHUNK

    idx3 = indices.reshape(NW, n_chunks, CHUNK)

    @functools.partial(
        pl.kernel,
        mesh=plsc.VectorSubcoreMesh(core_axis_name="c", subcore_axis_name="s"),
        out_type=jax.ShapeDtypeStruct((N, D), jnp.float32),
        scratch_types=[
            pltpu.VMEM((n_chunks, CHUNK), jnp.int32),
            pltpu.VMEM((CHUNK, D), jnp.float32),
            pltpu.SemaphoreType.DMA,
        ],
    )
    def gather_kernel(idx_hbm, table_hbm, out_hbm, idx_v, rows_v, sem):
        wid = lax.axis_index("s") * NUM_CORES + lax.axis_index("c")
        pltpu.sync_copy(idx_hbm.at[wid], idx_v)
        out_base = wid * per_w

        def body(j, carry):
            pltpu.async_copy(table_hbm.at[idx_v.at[j]], rows_v, sem).wait()
            pltpu.sync_copy(rows_v, out_hbm.at[pl.ds(out_base + j * CHUNK, CHUNK)])
            return carry

        lax.fori_loop(0, n_chunks, body, 0)

    out = gather_kernel(idx3, table)
    return out.reshape(B, L, D)


# speedup vs baseline: 1.6844x; 1.6844x over previous
import functools

import jax
import jax.numpy as jnp
from jax import lax
from jax.experimental import pallas as pl
from jax.experimental.pallas import tpu as pltpu
from jax.experimental.pallas import tpu_sc as plsc

NUM_CORES = 2
NUM_SUBCORES = 16
NW = NUM_CORES * NUM_SUBCORES
CHUNK = 128


def kernel(indices, table):
    B, L = indices.shape
    V, D = table.shape
    N = B * L
    assert N % (NW * CHUNK) == 0
    per_w = N // NW
    n_chunks = per_w // CHUNK

    idx3 = indices.reshape(NW, n_chunks, CHUNK)

    @functools.partial(
        pl.kernel,
        mesh=plsc.VectorSubcoreMesh(core_axis_name="c", subcore_axis_name="s"),
        out_type=jax.ShapeDtypeStruct((N, D), jnp.float32),
        scratch_types=[
            pltpu.VMEM((n_chunks, CHUNK), jnp.int32),
            pltpu.VMEM((CHUNK, D), jnp.float32),
            pltpu.SemaphoreType.DMA,
        ],
        compiler_params=pltpu.CompilerParams(use_tc_tiling_on_sc=False),
    )
    def gather_kernel(idx_hbm, table_hbm, out_hbm, idx_v, rows_v, sem):
        wid = lax.axis_index("s") * NUM_CORES + lax.axis_index("c")
        pltpu.sync_copy(idx_hbm.at[wid], idx_v)
        out_base = wid * per_w

        def body(j, carry):
            pltpu.async_copy(table_hbm.at[idx_v.at[j]], rows_v, sem).wait()
            pltpu.sync_copy(rows_v, out_hbm.at[pl.ds(out_base + j * CHUNK, CHUNK)])
            return carry

        lax.fori_loop(0, n_chunks, body, 0)

    out = gather_kernel(idx3, table)
    return out.reshape(B, L, D)


# speedup vs baseline: 1.8769x; 1.1143x over previous
import functools

import jax
import jax.numpy as jnp
from jax import lax
from jax.experimental import pallas as pl
from jax.experimental.pallas import tpu as pltpu
from jax.experimental.pallas import tpu_sc as plsc

NUM_CORES = 2
NUM_SUBCORES = 16
NW = NUM_CORES * NUM_SUBCORES
CHUNK = 128
K = 4


def kernel(indices, table):
    B, L = indices.shape
    V, D = table.shape
    N = B * L
    assert N % (NW * CHUNK * K) == 0
    per_w = N // NW
    n_chunks = per_w // CHUNK
    G = n_chunks // K
    assert G % 2 == 0
    rows_per_g = K * CHUNK

    idx3 = indices.reshape(NW, n_chunks, CHUNK)

    @functools.partial(
        pl.kernel,
        mesh=plsc.VectorSubcoreMesh(core_axis_name="c", subcore_axis_name="s"),
        out_type=jax.ShapeDtypeStruct((N, D), jnp.float32),
        scratch_types=[
            pltpu.VMEM((n_chunks, CHUNK), jnp.int32),
            pltpu.VMEM((rows_per_g, D), jnp.float32),
            pltpu.VMEM((rows_per_g, D), jnp.float32),
            pltpu.SemaphoreType.DMA,
            pltpu.SemaphoreType.DMA,
            pltpu.SemaphoreType.DMA,
            pltpu.SemaphoreType.DMA,
        ],
        compiler_params=pltpu.CompilerParams(use_tc_tiling_on_sc=False),
    )
    def gather_kernel(idx_hbm, table_hbm, out_hbm, idx_v, buf_a, buf_b,
                      gsem_a, gsem_b, osem_a, osem_b):
        wid = lax.axis_index("s") * NUM_CORES + lax.axis_index("c")
        pltpu.sync_copy(idx_hbm.at[wid], idx_v)
        out_base = wid * per_w

        bufs = (buf_a, buf_b)
        gsems = (gsem_a, gsem_b)
        osems = (osem_a, osem_b)

        def fire_gathers(g, s):
            for b in range(K):
                pltpu.async_copy(
                    table_hbm.at[idx_v.at[g * K + b]],
                    bufs[s].at[pl.ds(b * CHUNK, CHUNK)],
                    gsems[s],
                )

        def wait_gathers(g, s):
            for b in range(K):
                pltpu.make_async_copy(
                    table_hbm.at[idx_v.at[g * K + b]],
                    bufs[s].at[pl.ds(b * CHUNK, CHUNK)],
                    gsems[s],
                ).wait()

        def out_slice(g):
            return out_hbm.at[pl.ds(out_base + g * rows_per_g, rows_per_g)]

        def fire_out(g, s):
            pltpu.async_copy(bufs[s], out_slice(g), osems[s])

        def wait_out(g, s):
            pltpu.make_async_copy(bufs[s], out_slice(g), osems[s]).wait()

        fire_gathers(0, 0)

        def pair_body(p, carry):
            for s in (0, 1):
                g = 2 * p + s
                s_next = 1 - s

                @pl.when(g >= 1)
                def _():
                    wait_out(g - 1, s_next)

                @pl.when(g < G - 1)
                def _():
                    fire_gathers(g + 1, s_next)

                wait_gathers(g, s)
                fire_out(g, s)
            return carry

        lax.fori_loop(0, G // 2, pair_body, 0)
        wait_out(G - 1, (G - 1) % 2)

    out = gather_kernel(idx3, table)
    return out.reshape(B, L, D)
